# baseline (speedup 1.0000x reference)
import functools

import jax
import jax.numpy as jnp
from jax import lax
from jax.experimental import pallas as pl
from jax.experimental.pallas import tpu as pltpu
from jax.experimental.pallas import tpu_sc as plsc

_B, _C, _H, _W = 16, 256, 32, 32
_HW = _H * _W
_NC, _NS = 2, 16
_RPT = _HW // _NS
_BPC = _B // _NC


def _sc_body(row_hbm, col_hbm, out_hbm, blk, sem):
    c = lax.axis_index("c")
    s = lax.axis_index("s")
    pltpu.sync_copy(col_hbm.at[pl.ds(0, _W)], blk.at[pl.ds(0, _W), pl.ds(0, _C)])
    pltpu.sync_copy(col_hbm.at[pl.ds(0, _W)], blk.at[pl.ds(_W, _W), pl.ds(0, _C)])
    h0 = s * 2
    for half in range(2):
        for q in range(2):
            idx = jnp.full((16,), h0 + half, jnp.int32)
            pltpu.sync_copy(
                row_hbm.at[idx],
                blk.at[pl.ds(half * _W + q * 16, 16), pl.ds(_C, _C)],
            )
    r0 = s * _RPT
    copies = [
        pltpu.async_copy(blk, out_hbm.at[c * _BPC + i, pl.ds(r0, _RPT)], sem)
        for i in range(_BPC)
    ]
    for cp in copies:
        cp.wait()


def kernel(mask, row_embed, col_embed):
    B, H, W = mask.shape
    C = row_embed.shape[1]
    mesh = plsc.VectorSubcoreMesh(
        core_axis_name="c", subcore_axis_name="s",
        num_cores=_NC, num_subcores=_NS,
    )
    sc_call = functools.partial(
        pl.kernel,
        out_type=jax.ShapeDtypeStruct((B, H * W, 2 * C), jnp.float32),
        mesh=mesh,
        scratch_types=[
            pltpu.VMEM((_RPT, 2 * _C), jnp.float32),
            pltpu.SemaphoreType.DMA,
        ],
    )(_sc_body)
    out = sc_call(row_embed, col_embed)
    return out.reshape(B, H, W, 2 * C).transpose(0, 3, 1, 2)


# speedup vs baseline: 4.1385x; 4.1385x over previous
import jax
import jax.numpy as jnp
from jax import lax
from jax.experimental import pallas as pl
from jax.experimental.pallas import tpu as pltpu

_B, _C, _H, _W = 16, 256, 32, 32
_HW = _H * _W


def _pos_kernel(row_ref, col_ref, out_ref, plane, sem):
    col = col_ref[:_W, :]
    row = row_ref[:_H, :]
    plane[:, :_C] = jnp.broadcast_to(
        col[None, :, :], (_H, _W, _C)).reshape(_HW, _C)
    plane[:, _C:] = jnp.broadcast_to(
        row[:, None, :], (_H, _W, _C)).reshape(_HW, _C)
    copies = [pltpu.make_async_copy(plane, out_ref.at[b], sem) for b in range(_B)]
    for cp in copies:
        cp.start()
    for cp in copies:
        cp.wait()


def kernel(mask, row_embed, col_embed):
    B, H, W = mask.shape
    C = row_embed.shape[1]
    out = pl.pallas_call(
        _pos_kernel,
        in_specs=[
            pl.BlockSpec(memory_space=pltpu.VMEM),
            pl.BlockSpec(memory_space=pltpu.VMEM),
        ],
        out_specs=pl.BlockSpec(memory_space=pl.ANY),
        out_shape=jax.ShapeDtypeStruct((B, H * W, 2 * C), jnp.float32),
        scratch_shapes=[
            pltpu.VMEM((H * W, 2 * C), jnp.float32),
            pltpu.SemaphoreType.DMA,
        ],
    )(row_embed, col_embed)
    return out.reshape(B, H, W, 2 * C).transpose(0, 3, 1, 2)


# speedup vs baseline: 4.2294x; 1.0219x over previous
import jax
import jax.numpy as jnp
from jax import lax
from jax.experimental import pallas as pl
from jax.experimental.pallas import tpu as pltpu

_B, _C, _H, _W = 16, 256, 32, 32
_HW = _H * _W
_NCHUNK = 4
_RPC = _HW // _NCHUNK
_HPC = _H // _NCHUNK


def _pos_kernel(row_ref, col_ref, out_ref, plane, sem):
    col = col_ref[:_W, :]
    row = row_ref[:_H, :]
    copies = []
    for q in range(_NCHUNK):
        r0 = q * _RPC
        plane[pl.ds(r0, _RPC), :_C] = jnp.broadcast_to(
            col[None, :, :], (_HPC, _W, _C)).reshape(_RPC, _C)
        plane[pl.ds(r0, _RPC), _C:] = jnp.broadcast_to(
            row[q * _HPC:(q + 1) * _HPC, None, :], (_HPC, _W, _C)
        ).reshape(_RPC, _C)
        chunk = plane.at[pl.ds(r0, _RPC)]
        for b in range(_B):
            cp = pltpu.make_async_copy(
                chunk, out_ref.at[b, pl.ds(r0, _RPC)], sem)
            cp.start()
            copies.append(cp)
    for cp in copies:
        cp.wait()


def kernel(mask, row_embed, col_embed):
    B, H, W = mask.shape
    C = row_embed.shape[1]
    out = pl.pallas_call(
        _pos_kernel,
        in_specs=[
            pl.BlockSpec(memory_space=pltpu.VMEM),
            pl.BlockSpec(memory_space=pltpu.VMEM),
        ],
        out_specs=pl.BlockSpec(memory_space=pl.ANY),
        out_shape=jax.ShapeDtypeStruct((B, H * W, 2 * C), jnp.float32),
        scratch_shapes=[
            pltpu.VMEM((H * W, 2 * C), jnp.float32),
            pltpu.SemaphoreType.DMA,
        ],
    )(row_embed, col_embed)
    return out.reshape(B, H, W, 2 * C).transpose(0, 3, 1, 2)
